# baseline (speedup 1.0000x reference)
import functools

import jax
import jax.numpy as jnp
from jax import lax
from jax.experimental import pallas as pl
from jax.experimental.pallas import tpu as pltpu
from jax.experimental.pallas import tpu_sc as plsc

N = 43648
C = 16
K = 436
R = 341
L = 128

NR_BIG = 1368
NR_SMALL = N - 31 * NR_BIG
NG = NR_BIG // 16 + 1


def _sc_rowmax_body(cls_hbm, m_hbm, cls_v, m_v):
    nc = 2
    wid = lax.axis_index("s") * nc + lax.axis_index("c")
    base = wid * NR_BIG
    is_small = wid == 31
    nr = jnp.where(is_small, NR_SMALL, NR_BIG)

    @pl.when(jnp.logical_not(is_small))
    def _():
        pltpu.sync_copy(cls_hbm.at[pl.ds(base * C, NR_BIG * C)], cls_v)

    @pl.when(is_small)
    def _():
        pltpu.sync_copy(cls_hbm.at[pl.ds(base * C, NR_SMALL * C)],
                        cls_v.at[pl.ds(0, NR_SMALL * C)])

    lane = lax.iota(jnp.int32, 16)
    last = nr - 1

    def group(g, carry):
        rows = jnp.minimum(g * 16 + lane, last)
        flat = rows * C
        m = plsc.load_gather(cls_v, [flat + lane])
        for j in range(1, C):
            cols = lax.rem(lane + j, jnp.int32(C))
            m = jnp.maximum(m, plsc.load_gather(cls_v, [flat + cols]))
        plsc.store_scatter(m_v, [rows], m)
        return carry

    lax.fori_loop(0, NG, group, jnp.int32(0))

    @pl.when(jnp.logical_not(is_small))
    def _():
        pltpu.sync_copy(m_v, m_hbm.at[pl.ds(base, NR_BIG)])

    @pl.when(is_small)
    def _():
        pltpu.sync_copy(m_v.at[pl.ds(0, NR_SMALL)],
                        m_hbm.at[pl.ds(base, NR_SMALL)])


_sc_rowmax = functools.partial(
    pl.kernel,
    mesh=plsc.VectorSubcoreMesh(core_axis_name="c", subcore_axis_name="s"),
    out_type=jax.ShapeDtypeStruct((N,), jnp.float32),
    scratch_types=[
        pltpu.VMEM((NR_BIG * C,), jnp.float32),
        pltpu.VMEM((NR_BIG,), jnp.float32),
    ],
    compiler_params=pltpu.CompilerParams(needs_layout_passes=False),
)(_sc_rowmax_body)


def _tc_body(m_ref, cent_ref, pos_ref, neg_ref, w_ref, fg_ref, sdps_ref):
    m = m_ref[...]
    scores = jax.nn.sigmoid(m)
    cent = cent_ref[...]
    w_ref[...] = jax.nn.sigmoid(cent) * scores
    sdps_ref[0, 0] = jnp.sum(scores) * (1.0 / N)

    bits = jax.lax.bitcast_convert_type(scores, jnp.int32)
    nbits = jnp.int32(0x3F800000) - bits
    row = jax.lax.broadcasted_iota(jnp.int32, (R, L), 0)
    col = jax.lax.broadcasted_iota(jnp.int32, (R, L), 1)
    key2 = (N - 1) - (row * L + col)

    one = jnp.int32(1)
    hi = jnp.int32(1 << 16)
    zero = jnp.int32(0)

    def packed_count(mp, mn):
        s = jnp.sum(jnp.where(mp, one, zero) + jnp.where(mn, hi, zero))
        return s & jnp.int32(0xFFFF), jax.lax.shift_right_logical(s, 16)

    def vstep(i, carry):
        tp, tn = carry
        bit = jnp.left_shift(one, 29 - i)
        cp = tp | bit
        cn = tn | bit
        cntp, cntn = packed_count(bits >= cp, nbits >= cn)
        return (jnp.where(cntp >= K, cp, tp), jnp.where(cntn >= K, cn, tn))

    tp, tn = jax.lax.fori_loop(0, 30, vstep, (zero, zero))
    tied_p = bits == tp
    tied_n = nbits == tn
    gt_p = bits > tp
    gt_n = nbits > tn
    cgtp, cgtn = packed_count(gt_p, gt_n)
    ctp, ctn = packed_count(tied_p, tied_n)
    need_p = K - cgtp
    need_n = K - cgtn

    def no_ties():
        return zero, zero

    def with_ties():
        def tstep(i, carry):
            jp, jn = carry
            bit = jnp.left_shift(one, 15 - i)
            cp = jp | bit
            cn = jn | bit
            cntp, cntn = packed_count(tied_p & (key2 >= cp),
                                      tied_n & (key2 >= cn))
            return (jnp.where(cntp >= need_p, cp, jp),
                    jnp.where(cntn >= need_n, cn, jn))
        return jax.lax.fori_loop(0, 16, tstep, (zero, zero))

    jp, jn = jax.lax.cond(
        (ctp == need_p) & (ctn == need_n), no_ties, with_ties)
    pos_sel = gt_p | (tied_p & (key2 >= jp))
    neg_sel = gt_n | (tied_n & (key2 >= jn))
    fg_ref[0, 0] = jnp.sum(jnp.where(pos_sel, scores, 0.0))
    pos_ref[...] = pos_sel & jnp.logical_not(neg_sel)
    neg_ref[...] = neg_sel


def kernel(t_cls_scores, t_bbox_preds, t_centernesses):
    del t_bbox_preds
    m = _sc_rowmax(t_cls_scores.reshape(N * C))
    cent = t_centernesses.reshape(R, L)
    pos, neg, w, fg, sdps = pl.pallas_call(
        _tc_body,
        out_shape=[
            jax.ShapeDtypeStruct((R, L), jnp.bool_),
            jax.ShapeDtypeStruct((R, L), jnp.bool_),
            jax.ShapeDtypeStruct((R, L), jnp.float32),
            jax.ShapeDtypeStruct((1, 1), jnp.float32),
            jax.ShapeDtypeStruct((1, 1), jnp.float32),
        ],
        out_specs=[
            pl.BlockSpec(memory_space=pltpu.VMEM),
            pl.BlockSpec(memory_space=pltpu.VMEM),
            pl.BlockSpec(memory_space=pltpu.VMEM),
            pl.BlockSpec(memory_space=pltpu.SMEM),
            pl.BlockSpec(memory_space=pltpu.SMEM),
        ],
    )(m.reshape(R, L), cent)
    return (
        pos.reshape(N),
        neg.reshape(N),
        w.reshape(N),
        fg[0, 0],
        sdps[0, 0],
    )


# speedup vs baseline: 1.0007x; 1.0007x over previous
import functools

import jax
import jax.numpy as jnp
from jax import lax
from jax.experimental import pallas as pl
from jax.experimental.pallas import tpu as pltpu
from jax.experimental.pallas import tpu_sc as plsc

N = 43648
C = 16
K = 436
R = 341
L = 128

NR_BIG = 1368
NR_SMALL = N - 31 * NR_BIG
NG = NR_BIG // 16 + 1


def _sc_rowmax_body(cls_hbm, m_hbm, cls_v, m_v):
    nc = 2
    wid = lax.axis_index("s") * nc + lax.axis_index("c")
    base = wid * NR_BIG
    is_small = wid == 31
    nr = jnp.where(is_small, NR_SMALL, NR_BIG)

    @pl.when(jnp.logical_not(is_small))
    def _():
        pltpu.sync_copy(cls_hbm.at[pl.ds(base, NR_BIG)], cls_v)

    @pl.when(is_small)
    def _():
        pltpu.sync_copy(cls_hbm.at[pl.ds(base, NR_SMALL)],
                        cls_v.at[pl.ds(0, NR_SMALL)])

    lane = lax.iota(jnp.int32, 16)
    last = nr - 1

    def group(g, carry):
        rows = jnp.minimum(g * 16 + lane, last)
        m = plsc.load_gather(cls_v, [rows, lane])
        for j in range(1, C):
            cols = lax.rem(lane + j, jnp.int32(C))
            m = jnp.maximum(m, plsc.load_gather(cls_v, [rows, cols]))
        plsc.store_scatter(m_v, [rows], m)
        return carry

    lax.fori_loop(0, NG, group, jnp.int32(0))

    @pl.when(jnp.logical_not(is_small))
    def _():
        pltpu.sync_copy(m_v, m_hbm.at[pl.ds(base, NR_BIG)])

    @pl.when(is_small)
    def _():
        pltpu.sync_copy(m_v.at[pl.ds(0, NR_SMALL)],
                        m_hbm.at[pl.ds(base, NR_SMALL)])


_sc_rowmax = functools.partial(
    pl.kernel,
    mesh=plsc.VectorSubcoreMesh(core_axis_name="c", subcore_axis_name="s"),
    out_type=jax.ShapeDtypeStruct((N,), jnp.float32),
    scratch_types=[
        pltpu.VMEM((NR_BIG, C), jnp.float32),
        pltpu.VMEM((NR_BIG,), jnp.float32),
    ],
    compiler_params=pltpu.CompilerParams(
        needs_layout_passes=False, use_tc_tiling_on_sc=False),
)(_sc_rowmax_body)


def _tc_body(m_ref, cent_ref, pos_ref, neg_ref, w_ref, fg_ref, sdps_ref):
    m = m_ref[...]
    scores = jax.nn.sigmoid(m)
    cent = cent_ref[...]
    w_ref[...] = jax.nn.sigmoid(cent) * scores
    sdps_ref[0, 0] = jnp.sum(scores) * (1.0 / N)

    bits = jax.lax.bitcast_convert_type(scores, jnp.int32)
    nbits = jnp.int32(0x3F800000) - bits
    row = jax.lax.broadcasted_iota(jnp.int32, (R, L), 0)
    col = jax.lax.broadcasted_iota(jnp.int32, (R, L), 1)
    key2 = (N - 1) - (row * L + col)

    one = jnp.int32(1)
    hi = jnp.int32(1 << 16)
    zero = jnp.int32(0)

    def packed_count(mp, mn):
        s = jnp.sum(jnp.where(mp, one, zero) + jnp.where(mn, hi, zero))
        return s & jnp.int32(0xFFFF), jax.lax.shift_right_logical(s, 16)

    def vstep(i, carry):
        tp, tn = carry
        bit = jnp.left_shift(one, 29 - i)
        cp = tp | bit
        cn = tn | bit
        cntp, cntn = packed_count(bits >= cp, nbits >= cn)
        return (jnp.where(cntp >= K, cp, tp), jnp.where(cntn >= K, cn, tn))

    tp, tn = jax.lax.fori_loop(0, 30, vstep, (zero, zero))
    tied_p = bits == tp
    tied_n = nbits == tn
    gt_p = bits > tp
    gt_n = nbits > tn
    cgtp, cgtn = packed_count(gt_p, gt_n)
    ctp, ctn = packed_count(tied_p, tied_n)
    need_p = K - cgtp
    need_n = K - cgtn

    def no_ties():
        return zero, zero

    def with_ties():
        def tstep(i, carry):
            jp, jn = carry
            bit = jnp.left_shift(one, 15 - i)
            cp = jp | bit
            cn = jn | bit
            cntp, cntn = packed_count(tied_p & (key2 >= cp),
                                      tied_n & (key2 >= cn))
            return (jnp.where(cntp >= need_p, cp, jp),
                    jnp.where(cntn >= need_n, cn, jn))
        return jax.lax.fori_loop(0, 16, tstep, (zero, zero))

    jp, jn = jax.lax.cond(
        (ctp == need_p) & (ctn == need_n), no_ties, with_ties)
    pos_sel = gt_p | (tied_p & (key2 >= jp))
    neg_sel = gt_n | (tied_n & (key2 >= jn))
    fg_ref[0, 0] = jnp.sum(jnp.where(pos_sel, scores, 0.0))
    pos_ref[...] = pos_sel & jnp.logical_not(neg_sel)
    neg_ref[...] = neg_sel


def kernel(t_cls_scores, t_bbox_preds, t_centernesses):
    del t_bbox_preds
    m = _sc_rowmax(t_cls_scores)
    cent = t_centernesses.reshape(R, L)
    pos, neg, w, fg, sdps = pl.pallas_call(
        _tc_body,
        out_shape=[
            jax.ShapeDtypeStruct((R, L), jnp.bool_),
            jax.ShapeDtypeStruct((R, L), jnp.bool_),
            jax.ShapeDtypeStruct((R, L), jnp.float32),
            jax.ShapeDtypeStruct((1, 1), jnp.float32),
            jax.ShapeDtypeStruct((1, 1), jnp.float32),
        ],
        out_specs=[
            pl.BlockSpec(memory_space=pltpu.VMEM),
            pl.BlockSpec(memory_space=pltpu.VMEM),
            pl.BlockSpec(memory_space=pltpu.VMEM),
            pl.BlockSpec(memory_space=pltpu.SMEM),
            pl.BlockSpec(memory_space=pltpu.SMEM),
        ],
    )(m.reshape(R, L), cent)
    return (
        pos.reshape(N),
        neg.reshape(N),
        w.reshape(N),
        fg[0, 0],
        sdps[0, 0],
    )


# speedup vs baseline: 3.5897x; 3.5873x over previous
import jax
import jax.numpy as jnp
from jax.experimental import pallas as pl
from jax.experimental.pallas import tpu as pltpu

N = 43648
C = 16
K = 436
R = 341
L = 128


def _body(cls_ref, cent_ref, pos_ref, neg_ref, w_ref, fg_ref, sdps_ref):
    x = cls_ref[...]
    m = jnp.max(x, axis=0)
    scores = jax.nn.sigmoid(m)
    cent = cent_ref[...]
    w_ref[...] = jax.nn.sigmoid(cent) * scores
    sdps_ref[0, 0] = jnp.sum(scores) * (1.0 / N)

    bits = jax.lax.bitcast_convert_type(scores, jnp.int32)
    nbits = jnp.int32(0x3F800000) - bits
    row = jax.lax.broadcasted_iota(jnp.int32, (R, L), 0)
    col = jax.lax.broadcasted_iota(jnp.int32, (R, L), 1)
    key2 = (N - 1) - (row * L + col)

    one = jnp.int32(1)
    hi = jnp.int32(1 << 16)
    zero = jnp.int32(0)

    def packed_count(mp, mn):
        s = jnp.sum(jnp.where(mp, one, zero) + jnp.where(mn, hi, zero))
        return s & jnp.int32(0xFFFF), jax.lax.shift_right_logical(s, 16)

    def vstep(i, carry):
        tp, tn = carry
        bit = jnp.left_shift(one, 29 - i)
        cp = tp | bit
        cn = tn | bit
        cntp, cntn = packed_count(bits >= cp, nbits >= cn)
        return (jnp.where(cntp >= K, cp, tp), jnp.where(cntn >= K, cn, tn))

    tp, tn = jax.lax.fori_loop(0, 30, vstep, (zero, zero))
    tied_p = bits == tp
    tied_n = nbits == tn
    gt_p = bits > tp
    gt_n = nbits > tn
    cgtp, cgtn = packed_count(gt_p, gt_n)
    ctp, ctn = packed_count(tied_p, tied_n)
    need_p = K - cgtp
    need_n = K - cgtn

    def no_ties():
        return zero, zero

    def with_ties():
        def tstep(i, carry):
            jp, jn = carry
            bit = jnp.left_shift(one, 15 - i)
            cp = jp | bit
            cn = jn | bit
            cntp, cntn = packed_count(tied_p & (key2 >= cp),
                                      tied_n & (key2 >= cn))
            return (jnp.where(cntp >= need_p, cp, jp),
                    jnp.where(cntn >= need_n, cn, jn))
        return jax.lax.fori_loop(0, 16, tstep, (zero, zero))

    jp, jn = jax.lax.cond(
        (ctp == need_p) & (ctn == need_n), no_ties, with_ties)
    pos_sel = gt_p | (tied_p & (key2 >= jp))
    neg_sel = gt_n | (tied_n & (key2 >= jn))
    fg_ref[0, 0] = jnp.sum(jnp.where(pos_sel, scores, 0.0))
    pos_ref[...] = pos_sel & jnp.logical_not(neg_sel)
    neg_ref[...] = neg_sel


def kernel(t_cls_scores, t_bbox_preds, t_centernesses):
    del t_bbox_preds
    x_t = t_cls_scores.T.reshape(C, R, L)
    cent = t_centernesses.reshape(R, L)
    pos, neg, w, fg, sdps = pl.pallas_call(
        _body,
        out_shape=[
            jax.ShapeDtypeStruct((R, L), jnp.bool_),
            jax.ShapeDtypeStruct((R, L), jnp.bool_),
            jax.ShapeDtypeStruct((R, L), jnp.float32),
            jax.ShapeDtypeStruct((1, 1), jnp.float32),
            jax.ShapeDtypeStruct((1, 1), jnp.float32),
        ],
        out_specs=[
            pl.BlockSpec(memory_space=pltpu.VMEM),
            pl.BlockSpec(memory_space=pltpu.VMEM),
            pl.BlockSpec(memory_space=pltpu.VMEM),
            pl.BlockSpec(memory_space=pltpu.SMEM),
            pl.BlockSpec(memory_space=pltpu.SMEM),
        ],
    )(x_t, cent)
    return (
        pos.reshape(N),
        neg.reshape(N),
        w.reshape(N),
        fg[0, 0],
        sdps[0, 0],
    )


# speedup vs baseline: 5.5754x; 1.5531x over previous
import jax
import jax.numpy as jnp
from jax.experimental import pallas as pl
from jax.experimental.pallas import tpu as pltpu

N = 43648
C = 16
K = 436
R = 341
L = 128


def _body(cls_ref, cent_ref, pos_ref, neg_ref, w_ref, fg_ref, sdps_ref):
    x = cls_ref[...]
    m = jnp.max(x, axis=0)
    scores = jax.nn.sigmoid(m)
    cent = cent_ref[...]
    w_ref[...] = jax.nn.sigmoid(cent) * scores
    sdps_ref[0, 0] = jnp.sum(scores) * (1.0 / N)

    bits = jax.lax.bitcast_convert_type(scores, jnp.int32)
    nbits = jnp.int32(0x3F800000) - bits
    row = jax.lax.broadcasted_iota(jnp.int32, (R, L), 0)
    col = jax.lax.broadcasted_iota(jnp.int32, (R, L), 1)
    key2 = (N - 1) - (row * L + col)

    one = jnp.int32(1)
    hi = jnp.int32(1 << 16)
    zero = jnp.int32(0)

    def packed_count(mp, mn):
        s = jnp.sum(jnp.where(mp, one, zero) + jnp.where(mn, hi, zero))
        return s & jnp.int32(0xFFFF), jax.lax.shift_right_logical(s, 16)

    def vstep(i, carry):
        tp, tn = carry
        bit = jnp.left_shift(one, 29 - i)
        cp = tp | bit
        cn = tn | bit
        cntp, cntn = packed_count(bits >= cp, nbits >= cn)
        return (jnp.where(cntp >= K, cp, tp), jnp.where(cntn >= K, cn, tn))

    tp, tn = jax.lax.fori_loop(0, 30, vstep, (zero, zero))
    tied_p = bits == tp
    tied_n = nbits == tn
    gt_p = bits > tp
    gt_n = nbits > tn
    cgtp, cgtn = packed_count(gt_p, gt_n)
    ctp, ctn = packed_count(tied_p, tied_n)
    need_p = K - cgtp
    need_n = K - cgtn

    def no_ties():
        return zero, zero

    def with_ties():
        def tstep(i, carry):
            jp, jn = carry
            bit = jnp.left_shift(one, 15 - i)
            cp = jp | bit
            cn = jn | bit
            cntp, cntn = packed_count(tied_p & (key2 >= cp),
                                      tied_n & (key2 >= cn))
            return (jnp.where(cntp >= need_p, cp, jp),
                    jnp.where(cntn >= need_n, cn, jn))
        return jax.lax.fori_loop(0, 16, tstep, (zero, zero))

    jp, jn = jax.lax.cond(
        (ctp == need_p) & (ctn == need_n), no_ties, with_ties)
    pos_sel = gt_p | (tied_p & (key2 >= jp))
    neg_sel = gt_n | (tied_n & (key2 >= jn))
    fg_ref[0, 0] = jnp.sum(jnp.where(pos_sel, scores, 0.0))
    pos_ref[...] = pos_sel & jnp.logical_not(neg_sel)
    neg_ref[...] = neg_sel


def kernel(t_cls_scores, t_bbox_preds, t_centernesses):
    del t_bbox_preds
    x_t = t_cls_scores.T.reshape(C, R, L)
    cent = t_centernesses.reshape(R, L)
    pos, neg, w, fg, sdps = pl.pallas_call(
        _body,
        out_shape=[
            jax.ShapeDtypeStruct((R, L), jnp.bool_),
            jax.ShapeDtypeStruct((R, L), jnp.bool_),
            jax.ShapeDtypeStruct((R, L), jnp.float32),
            jax.ShapeDtypeStruct((1, 1), jnp.float32),
            jax.ShapeDtypeStruct((1, 1), jnp.float32),
        ],
        out_specs=[
            pl.BlockSpec(memory_space=pltpu.VMEM),
            pl.BlockSpec(memory_space=pltpu.VMEM),
            pl.BlockSpec(memory_space=pltpu.VMEM),
            pl.BlockSpec(memory_space=pltpu.SMEM),
            pl.BlockSpec(memory_space=pltpu.SMEM),
        ],
        compiler_params=pltpu.CompilerParams(
            allow_input_fusion=[True, True]),
    )(x_t, cent)
    return (
        pos.reshape(N),
        neg.reshape(N),
        w.reshape(N),
        fg[0, 0],
        sdps[0, 0],
    )
